# baseline (speedup 1.0000x reference)
import functools

import jax
import jax.numpy as jnp
from jax import lax
from jax.experimental import pallas as pl
from jax.experimental.pallas import tpu as pltpu
from jax.experimental.pallas import tpu_sc as plsc

N_DIM = 32
E_DIM = 16
M_DIM = 32
HID = 128
N_EDGE = 160000
N_NODE = 10000

BLK = 2048
GRP = 128
N_WORKER = 32
G_PER_W = 40
N_GRP_PAD = N_WORKER * G_PER_W
N_EDGE_PAD = N_GRP_PAD * GRP
CHUNK = 10
N_NODE_PAD = 10240
DUMMY = N_NODE
STRIPE = N_NODE_PAD // 16


def _mvw_body(et_ref, hwt_ref, w1t_ref, b1_ref, w2t_ref, b2m_ref, st_ref, out_ref):
    hw_t = hwt_ref[...]
    h_t = jnp.maximum(jnp.dot(w1t_ref[...], et_ref[...]) + b1_ref[...], 0.0)
    a_t = jnp.dot(w2t_ref[...], h_t.astype(jnp.bfloat16),
                  preferred_element_type=jnp.float32).astype(jnp.bfloat16)
    prod_t = a_t * jnp.tile(hw_t.astype(jnp.bfloat16), (M_DIM, 1))
    m_t = jnp.dot(st_ref[...], prod_t, preferred_element_type=jnp.float32)
    m_t = m_t + jnp.dot(b2m_ref[...], hw_t)
    q_len = BLK // 4
    for q in range(4):
        out_ref[:, q * M_DIM:(q + 1) * M_DIM] = m_t[:, q * q_len:(q + 1) * q_len].T


def _compute_mvw(e_vw_t, h_w_t, W1t, b1, W2t, b2, sel_t):
    grid = (pl.cdiv(N_EDGE, BLK),)
    return pl.pallas_call(
        _mvw_body,
        grid=grid,
        in_specs=[
            pl.BlockSpec((E_DIM, BLK), lambda i: (0, i)),
            pl.BlockSpec((N_DIM, BLK), lambda i: (0, i)),
            pl.BlockSpec((HID, E_DIM), lambda i: (0, 0)),
            pl.BlockSpec((HID, 1), lambda i: (0, 0)),
            pl.BlockSpec((N_DIM * M_DIM, HID), lambda i: (0, 0)),
            pl.BlockSpec((M_DIM, N_DIM), lambda i: (0, 0)),
            pl.BlockSpec((M_DIM, N_DIM * M_DIM), lambda i: (0, 0)),
        ],
        out_specs=pl.BlockSpec((BLK // 4, 4 * M_DIM), lambda i: (i, 0)),
        out_shape=jax.ShapeDtypeStruct((N_EDGE_PAD // 4, 4 * M_DIM), jnp.float32),
        compiler_params=pltpu.CompilerParams(
            dimension_semantics=("arbitrary",),
        ),
    )(e_vw_t, h_w_t, W1t, b1.reshape(HID, 1), W2t, b2.reshape(M_DIM, N_DIM),
      sel_t)


def _make_scatter():
    mesh = plsc.VectorSubcoreMesh(core_axis_name="c", subcore_axis_name="s")

    @functools.partial(
        pl.kernel,
        mesh=mesh,
        out_type=jax.ShapeDtypeStruct((2, N_NODE_PAD, M_DIM), jnp.float32),
        scratch_types=[
            pltpu.VMEM((G_PER_W, GRP), jnp.int32),
            pltpu.VMEM((CHUNK * GRP, M_DIM), jnp.float32),
            pltpu.VMEM_SHARED((N_NODE_PAD, M_DIM), jnp.float32),
        ],
        compiler_params=pltpu.CompilerParams(use_tc_tiling_on_sc=False),
    )
    def scatter(mvw_hbm, idx_hbm, zero_hbm, out_hbm, idx_v, rows_v, acc_sh):
        c = lax.axis_index("c")
        s = lax.axis_index("s")
        wid = s * 2 + c
        base_g = wid * G_PER_W

        pltpu.sync_copy(zero_hbm.at[pl.ds(s * STRIPE, STRIPE)],
                        acc_sh.at[pl.ds(s * STRIPE, STRIPE)])

        pltpu.sync_copy(idx_hbm.at[pl.ds(base_g, G_PER_W)], idx_v)

        plsc.subcore_barrier()

        def chunk_body(t, _):
            pltpu.sync_copy(
                mvw_hbm.at[pl.ds((base_g + t * CHUNK) * GRP, CHUNK * GRP)],
                rows_v)
            for j in range(CHUNK):
                pltpu.sync_copy(rows_v.at[pl.ds(j * GRP, GRP)],
                                acc_sh.at[idx_v.at[t * CHUNK + j]],
                                add=True)
            return 0

        lax.fori_loop(0, G_PER_W // CHUNK, chunk_body, 0)

        plsc.subcore_barrier()

        pltpu.sync_copy(acc_sh.at[pl.ds(s * STRIPE, STRIPE)],
                        out_hbm.at[c, pl.ds(s * STRIPE, STRIPE)])

    return scatter


_scatter_kernel = _make_scatter()


def _combine_body(p_ref, o_ref):
    o_ref[...] = p_ref[0] + p_ref[1]


def _combine(partials):
    return pl.pallas_call(
        _combine_body,
        out_shape=jax.ShapeDtypeStruct((N_NODE_PAD, M_DIM), jnp.float32),
    )(partials)


def kernel(index_v, h_w, e_vw, n_node, W1, b1, W2, b2):
    sel_t = (jnp.arange(N_DIM * M_DIM, dtype=jnp.int32)[None, :] // N_DIM
             == jnp.arange(M_DIM, dtype=jnp.int32)[:, None]).astype(jnp.bfloat16)
    m_vw = _compute_mvw(e_vw.T, h_w.T, W1.T, b1, W2.T.astype(jnp.bfloat16),
                        b2, sel_t)
    m_vw = m_vw.reshape(N_EDGE_PAD, M_DIM)
    idx_pad = jnp.concatenate([
        index_v.astype(jnp.int32),
        jnp.full((N_EDGE_PAD - N_EDGE,), DUMMY, jnp.int32),
    ])
    idx_pad = (idx_pad.reshape(N_EDGE_PAD // BLK, 4, BLK // 4)
               .transpose(0, 2, 1).reshape(-1))
    idx2d = idx_pad.reshape(N_GRP_PAD, GRP)
    zeros = jnp.zeros((N_NODE_PAD, M_DIM), jnp.float32)
    partials = _scatter_kernel(m_vw, idx2d, zeros)
    return _combine(partials)[:N_NODE]


# speedup vs baseline: 1.0307x; 1.0307x over previous
import functools

import jax
import jax.numpy as jnp
from jax import lax
from jax.experimental import pallas as pl
from jax.experimental.pallas import tpu as pltpu
from jax.experimental.pallas import tpu_sc as plsc

N_DIM = 32
E_DIM = 16
M_DIM = 32
HID = 128
N_EDGE = 160000
N_NODE = 10000

BLK = 4096
GRP = 128
N_WORKER = 32
G_PER_W = 40
N_GRP_PAD = N_WORKER * G_PER_W
N_EDGE_PAD = N_GRP_PAD * GRP
CHUNK = 10
N_NODE_PAD = 10240
DUMMY = N_NODE
STRIPE = N_NODE_PAD // 16


def _mvw_body(et_ref, hwt_ref, w1t_ref, b1_ref, w2t_ref, b2m_ref, st_ref, out_ref):
    hw_t = hwt_ref[...]
    h_t = jnp.maximum(jnp.dot(w1t_ref[...], et_ref[...]) + b1_ref[...], 0.0)
    a_t = jnp.dot(w2t_ref[...], h_t.astype(jnp.bfloat16),
                  preferred_element_type=jnp.float32).astype(jnp.bfloat16)
    prod_t = a_t * jnp.tile(hw_t.astype(jnp.bfloat16), (M_DIM, 1))
    m_t = jnp.dot(st_ref[...], prod_t, preferred_element_type=jnp.float32)
    m_t = m_t + jnp.dot(b2m_ref[...], hw_t)
    q_len = BLK // 4
    for q in range(4):
        out_ref[:, q * M_DIM:(q + 1) * M_DIM] = m_t[:, q * q_len:(q + 1) * q_len].T


def _compute_mvw(e_vw_t, h_w_t, W1t, b1, W2t, b2, sel_t):
    grid = (pl.cdiv(N_EDGE, BLK),)
    return pl.pallas_call(
        _mvw_body,
        grid=grid,
        in_specs=[
            pl.BlockSpec((E_DIM, BLK), lambda i: (0, i)),
            pl.BlockSpec((N_DIM, BLK), lambda i: (0, i)),
            pl.BlockSpec((HID, E_DIM), lambda i: (0, 0)),
            pl.BlockSpec((HID, 1), lambda i: (0, 0)),
            pl.BlockSpec((N_DIM * M_DIM, HID), lambda i: (0, 0)),
            pl.BlockSpec((M_DIM, N_DIM), lambda i: (0, 0)),
            pl.BlockSpec((M_DIM, N_DIM * M_DIM), lambda i: (0, 0)),
        ],
        out_specs=pl.BlockSpec((BLK // 4, 4 * M_DIM), lambda i: (i, 0)),
        out_shape=jax.ShapeDtypeStruct((N_EDGE_PAD // 4, 4 * M_DIM), jnp.float32),
        compiler_params=pltpu.CompilerParams(
            dimension_semantics=("arbitrary",),
        ),
    )(e_vw_t, h_w_t, W1t, b1.reshape(HID, 1), W2t, b2.reshape(M_DIM, N_DIM),
      sel_t)


def _make_scatter():
    mesh = plsc.VectorSubcoreMesh(core_axis_name="c", subcore_axis_name="s")

    @functools.partial(
        pl.kernel,
        mesh=mesh,
        out_type=jax.ShapeDtypeStruct((2, N_NODE_PAD, M_DIM), jnp.float32),
        scratch_types=[
            pltpu.VMEM((G_PER_W, GRP), jnp.int32),
            pltpu.VMEM((CHUNK * GRP, M_DIM), jnp.float32),
            pltpu.VMEM_SHARED((N_NODE_PAD, M_DIM), jnp.float32),
        ],
        compiler_params=pltpu.CompilerParams(use_tc_tiling_on_sc=False),
    )
    def scatter(mvw_hbm, idx_hbm, zero_hbm, out_hbm, idx_v, rows_v, acc_sh):
        c = lax.axis_index("c")
        s = lax.axis_index("s")
        wid = s * 2 + c
        base_g = wid * G_PER_W

        pltpu.sync_copy(zero_hbm.at[pl.ds(s * STRIPE, STRIPE)],
                        acc_sh.at[pl.ds(s * STRIPE, STRIPE)])

        pltpu.sync_copy(idx_hbm.at[pl.ds(base_g, G_PER_W)], idx_v)

        plsc.subcore_barrier()

        def chunk_body(t, _):
            pltpu.sync_copy(
                mvw_hbm.at[pl.ds((base_g + t * CHUNK) * GRP, CHUNK * GRP)],
                rows_v)
            for j in range(CHUNK):
                pltpu.sync_copy(rows_v.at[pl.ds(j * GRP, GRP)],
                                acc_sh.at[idx_v.at[t * CHUNK + j]],
                                add=True)
            return 0

        lax.fori_loop(0, G_PER_W // CHUNK, chunk_body, 0)

        plsc.subcore_barrier()

        pltpu.sync_copy(acc_sh.at[pl.ds(s * STRIPE, STRIPE)],
                        out_hbm.at[c, pl.ds(s * STRIPE, STRIPE)])

    return scatter


_scatter_kernel = _make_scatter()


def _combine_body(p_ref, o_ref):
    o_ref[...] = p_ref[0] + p_ref[1]


def _combine(partials):
    return pl.pallas_call(
        _combine_body,
        out_shape=jax.ShapeDtypeStruct((N_NODE_PAD, M_DIM), jnp.float32),
    )(partials)


def kernel(index_v, h_w, e_vw, n_node, W1, b1, W2, b2):
    sel_t = (jnp.arange(N_DIM * M_DIM, dtype=jnp.int32)[None, :] // N_DIM
             == jnp.arange(M_DIM, dtype=jnp.int32)[:, None]).astype(jnp.bfloat16)
    m_vw = _compute_mvw(e_vw.T, h_w.T, W1.T, b1, W2.T.astype(jnp.bfloat16),
                        b2, sel_t)
    m_vw = m_vw.reshape(N_EDGE_PAD, M_DIM)
    idx_pad = jnp.concatenate([
        index_v.astype(jnp.int32),
        jnp.full((N_EDGE_PAD - N_EDGE,), DUMMY, jnp.int32),
    ])
    idx_pad = (idx_pad.reshape(N_EDGE_PAD // BLK, 4, BLK // 4)
               .transpose(0, 2, 1).reshape(-1))
    idx2d = idx_pad.reshape(N_GRP_PAD, GRP)
    zeros = jnp.zeros((N_NODE_PAD, M_DIM), jnp.float32)
    partials = _scatter_kernel(m_vw, idx2d, zeros)
    return _combine(partials)[:N_NODE]


# speedup vs baseline: 1.0311x; 1.0004x over previous
import functools

import jax
import jax.numpy as jnp
from jax import lax
from jax.experimental import pallas as pl
from jax.experimental.pallas import tpu as pltpu
from jax.experimental.pallas import tpu_sc as plsc

N_DIM = 32
E_DIM = 16
M_DIM = 32
HID = 128
N_EDGE = 160000
N_NODE = 10000

BLK = 4096
GRP = 128
N_WORKER = 32
G_PER_W = 40
N_GRP_PAD = N_WORKER * G_PER_W
N_EDGE_PAD = N_GRP_PAD * GRP
CHUNK = 10
N_NODE_PAD = 10240
DUMMY = N_NODE
STRIPE = N_NODE_PAD // 16


def _mvw_body(et_ref, hwt_ref, w1t_ref, b1_ref, w2t_ref, b2m_ref, st_ref, out_ref):
    hw_t = hwt_ref[...]
    h_t = jnp.maximum(jnp.dot(w1t_ref[...], et_ref[...]) + b1_ref[...], 0.0)
    a_t = jnp.dot(w2t_ref[...], h_t.astype(jnp.bfloat16),
                  preferred_element_type=jnp.float32).astype(jnp.bfloat16)
    prod_t = a_t * jnp.tile(hw_t.astype(jnp.bfloat16), (M_DIM, 1))
    m_t = jnp.dot(st_ref[...], prod_t, preferred_element_type=jnp.float32)
    m_t = m_t + jnp.dot(b2m_ref[...], hw_t)
    q_len = BLK // 4
    for q in range(4):
        out_ref[:, q * M_DIM:(q + 1) * M_DIM] = m_t[:, q * q_len:(q + 1) * q_len].T


def _compute_mvw(e_vw_t, h_w_t, W1t, b1, W2t, b2, sel_t):
    grid = (pl.cdiv(N_EDGE, BLK),)
    return pl.pallas_call(
        _mvw_body,
        grid=grid,
        in_specs=[
            pl.BlockSpec((E_DIM, BLK), lambda i: (0, i)),
            pl.BlockSpec((N_DIM, BLK), lambda i: (0, i)),
            pl.BlockSpec((HID, E_DIM), lambda i: (0, 0)),
            pl.BlockSpec((HID, 1), lambda i: (0, 0)),
            pl.BlockSpec((N_DIM * M_DIM, HID), lambda i: (0, 0)),
            pl.BlockSpec((M_DIM, N_DIM), lambda i: (0, 0)),
            pl.BlockSpec((M_DIM, N_DIM * M_DIM), lambda i: (0, 0)),
        ],
        out_specs=pl.BlockSpec((BLK // 4, 4 * M_DIM), lambda i: (i, 0)),
        out_shape=jax.ShapeDtypeStruct((N_EDGE_PAD // 4, 4 * M_DIM), jnp.float32),
        compiler_params=pltpu.CompilerParams(
            dimension_semantics=("parallel",),
        ),
    )(e_vw_t, h_w_t, W1t, b1.reshape(HID, 1), W2t, b2.reshape(M_DIM, N_DIM),
      sel_t)


def _make_scatter():
    mesh = plsc.VectorSubcoreMesh(core_axis_name="c", subcore_axis_name="s")

    @functools.partial(
        pl.kernel,
        mesh=mesh,
        out_type=jax.ShapeDtypeStruct((2, N_NODE_PAD, M_DIM), jnp.float32),
        scratch_types=[
            pltpu.VMEM((G_PER_W, GRP), jnp.int32),
            pltpu.VMEM((CHUNK * GRP, M_DIM), jnp.float32),
            pltpu.VMEM_SHARED((N_NODE_PAD, M_DIM), jnp.float32),
        ],
        compiler_params=pltpu.CompilerParams(use_tc_tiling_on_sc=False),
    )
    def scatter(mvw_hbm, idx_hbm, zero_hbm, out_hbm, idx_v, rows_v, acc_sh):
        c = lax.axis_index("c")
        s = lax.axis_index("s")
        wid = s * 2 + c
        base_g = wid * G_PER_W

        pltpu.sync_copy(zero_hbm.at[pl.ds(s * STRIPE, STRIPE)],
                        acc_sh.at[pl.ds(s * STRIPE, STRIPE)])

        pltpu.sync_copy(idx_hbm.at[pl.ds(base_g, G_PER_W)], idx_v)

        plsc.subcore_barrier()

        def chunk_body(t, _):
            pltpu.sync_copy(
                mvw_hbm.at[pl.ds((base_g + t * CHUNK) * GRP, CHUNK * GRP)],
                rows_v)
            for j in range(CHUNK):
                pltpu.sync_copy(rows_v.at[pl.ds(j * GRP, GRP)],
                                acc_sh.at[idx_v.at[t * CHUNK + j]],
                                add=True)
            return 0

        lax.fori_loop(0, G_PER_W // CHUNK, chunk_body, 0)

        plsc.subcore_barrier()

        pltpu.sync_copy(acc_sh.at[pl.ds(s * STRIPE, STRIPE)],
                        out_hbm.at[c, pl.ds(s * STRIPE, STRIPE)])

    return scatter


_scatter_kernel = _make_scatter()


def _combine_body(p_ref, o_ref):
    o_ref[...] = p_ref[0] + p_ref[1]


def _combine(partials):
    return pl.pallas_call(
        _combine_body,
        out_shape=jax.ShapeDtypeStruct((N_NODE_PAD, M_DIM), jnp.float32),
    )(partials)


def kernel(index_v, h_w, e_vw, n_node, W1, b1, W2, b2):
    sel_t = (jnp.arange(N_DIM * M_DIM, dtype=jnp.int32)[None, :] // N_DIM
             == jnp.arange(M_DIM, dtype=jnp.int32)[:, None]).astype(jnp.bfloat16)
    m_vw = _compute_mvw(e_vw.T, h_w.T, W1.T, b1, W2.T.astype(jnp.bfloat16),
                        b2, sel_t)
    m_vw = m_vw.reshape(N_EDGE_PAD, M_DIM)
    idx_pad = jnp.concatenate([
        index_v.astype(jnp.int32),
        jnp.full((N_EDGE_PAD - N_EDGE,), DUMMY, jnp.int32),
    ])
    idx_pad = (idx_pad.reshape(N_EDGE_PAD // BLK, 4, BLK // 4)
               .transpose(0, 2, 1).reshape(-1))
    idx2d = idx_pad.reshape(N_GRP_PAD, GRP)
    zeros = jnp.zeros((N_NODE_PAD, M_DIM), jnp.float32)
    partials = _scatter_kernel(m_vw, idx2d, zeros)
    return _combine(partials)[:N_NODE]
